# baseline (speedup 1.0000x reference)
import functools

import jax
import jax.numpy as jnp
from jax import lax
from jax.experimental import pallas as pl
from jax.experimental.pallas import tpu as pltpu
from jax.experimental.pallas import tpu_sc as plsc

BATCH = 16384
N_FIELDS = 26
EMBED_DIM = 16
TABLE_ROWS = 1000000
T_PAD = 1000448

NC = 2
NS = 16
NW = NC * NS
B_PER_W = BATCH // NW
ITEMS_PER_CHUNK = 4
IDX_PER_CHUNK = ITEMS_PER_CHUNK * N_FIELDS
NCHUNKS = B_PER_W // ITEMS_PER_CHUNK
NBUF = 8
FLAT_PER_W = B_PER_W * N_FIELDS
OFF_TILE = 208
NVEC_OFF = OFF_TILE // EMBED_DIM
T_SLICE = T_PAD // NS

TC_BLK = 262144


def _tc_dot_kernel(tt_ref, w_ref, t_ref):
    t_ref[...] = jnp.sum(tt_ref[...] * (w_ref[...] * (1.0 / N_FIELDS)), axis=0)


def _sc_kernel(x_hbm, off_hbm, t_hbm, bias_hbm, out_hbm,
               t_sh, idx_v, off_v, b_v, acc_v,
               buf0, buf1, buf2, buf3, buf4, buf5, buf6, buf7,
               sem0, sem1, sem2, sem3, sem4, sem5, sem6, sem7, sem_stage):
    bufs = [buf0, buf1, buf2, buf3, buf4, buf5, buf6, buf7]
    sems = [sem0, sem1, sem2, sem3, sem4, sem5, sem6, sem7]

    sid = lax.axis_index("s")
    wid = sid * NC + lax.axis_index("c")
    base_flat = wid * FLAT_PER_W
    base_out = wid * B_PER_W

    sl = pl.ds(sid * T_SLICE, T_SLICE)
    stage_cp = pltpu.async_copy(t_hbm.at[sl], t_sh.at[sl], sem_stage)

    pltpu.sync_copy(x_hbm.at[pl.ds(base_flat, FLAT_PER_W)], idx_v)
    pltpu.sync_copy(off_hbm, off_v)
    pltpu.sync_copy(bias_hbm, b_v)

    off_regs = [off_v[pl.ds(k * EMBED_DIM, EMBED_DIM)] for k in range(NVEC_OFF)]

    def off_body(j, carry):
        p = j * OFF_TILE
        for k in range(NVEC_OFF):
            sl2 = pl.ds(p + k * EMBED_DIM, EMBED_DIM)
            idx_v[sl2] = idx_v[sl2] + off_regs[k]
        return carry

    lax.fori_loop(0, FLAT_PER_W // OFF_TILE, off_body, 0)

    stage_cp.wait()
    plsc.subcore_barrier()

    def gather_start(c, slot):
        idx_sl = idx_v.at[pl.ds(c * IDX_PER_CHUNK, IDX_PER_CHUNK)]
        pltpu.async_copy(t_sh.at[idx_sl], bufs[slot], sems[slot])

    def gather_wait(slot):
        idx_sl = idx_v.at[pl.ds(0, IDX_PER_CHUNK)]
        pltpu.make_async_copy(t_sh.at[idx_sl], bufs[slot], sems[slot]).wait()

    for b in range(NBUF):
        gather_start(b, b)

    lane_iota = lax.iota(jnp.int32, EMBED_DIM)
    WINDOWS = ((0, 0, 16, 10), (24, 2, 40, 12), (48, 4, 64, 14), (72, 6, 88, 16))

    ITEMS_PER_OUTER = NBUF * ITEMS_PER_CHUNK
    N_ACC = ITEMS_PER_OUTER // EMBED_DIM

    def outer(c0, carry):
        accs = [jnp.zeros((EMBED_DIM,), jnp.float32) for _ in range(N_ACC)]
        for b in range(NBUF):
            c = c0 * NBUF + b
            gather_wait(b)
            for item in range(ITEMS_PER_CHUNK):
                lo_off, lo_from, hi_off, hi_below = WINDOWS[item]
                v_lo = bufs[b][pl.ds(lo_off, EMBED_DIM)]
                v_hi = bufs[b][pl.ds(hi_off, EMBED_DIM)]
                if lo_from:
                    v_lo = jnp.where(lane_iota >= lo_from, v_lo, 0.0)
                if hi_below < EMBED_DIM:
                    v_hi = jnp.where(lane_iota < hi_below, v_hi, 0.0)
                v = v_lo + v_hi
                g = b * ITEMS_PER_CHUNK + item
                acc_i, lane = g // EMBED_DIM, g % EMBED_DIM
                accs[acc_i] = jnp.where(lane_iota == lane, jnp.sum(v), accs[acc_i])

            @pl.when(c + NBUF < NCHUNKS)
            def _():
                gather_start(c + NBUF, b)
        for i in range(N_ACC):
            acc_v[pl.ds(c0 * ITEMS_PER_OUTER + i * EMBED_DIM, EMBED_DIM)] = accs[i]
        return carry

    lax.fori_loop(0, NCHUNKS // NBUF, outer, 0)

    bv = b_v[...]

    def sig_body(v, carry):
        sl2 = pl.ds(v * EMBED_DIM, EMBED_DIM)
        z = acc_v[sl2] + bv
        acc_v[sl2] = 1.0 / (1.0 + jnp.exp(-z))
        return carry

    lax.fori_loop(0, B_PER_W // EMBED_DIM, sig_body, 0)

    pltpu.sync_copy(acc_v, out_hbm.at[pl.ds(base_out, B_PER_W)])


@jax.jit
def kernel(x, offsets, emb_table, W, b):
    x_flat = x.astype(jnp.int32).reshape(-1)
    off_tile = jnp.tile(offsets.astype(jnp.int32), OFF_TILE // N_FIELDS)
    b_vec = jnp.broadcast_to(b.astype(jnp.float32), (EMBED_DIM,))

    table_t = emb_table.T
    n_blk = (T_PAD + TC_BLK - 1) // TC_BLK
    t = pl.pallas_call(
        _tc_dot_kernel,
        grid=(n_blk,),
        in_specs=[
            pl.BlockSpec((EMBED_DIM, TC_BLK), lambda i: (0, i)),
            pl.BlockSpec((EMBED_DIM, 1), lambda i: (0, 0)),
        ],
        out_specs=pl.BlockSpec((TC_BLK,), lambda i: (i,)),
        out_shape=jax.ShapeDtypeStruct((T_PAD,), jnp.float32),
    )(table_t, W.astype(jnp.float32))

    mesh = plsc.VectorSubcoreMesh(core_axis_name="c", subcore_axis_name="s")
    run = pl.kernel(
        _sc_kernel,
        mesh=mesh,
        out_type=jax.ShapeDtypeStruct((BATCH,), jnp.float32),
        compiler_params=pltpu.CompilerParams(
            needs_layout_passes=False,
            use_tc_tiling_on_sc=False,
            skip_device_barrier=True,
        ),
        scratch_types=[
            pltpu.VMEM_SHARED((T_PAD,), jnp.float32),
            pltpu.VMEM((FLAT_PER_W,), jnp.int32),
            pltpu.VMEM((OFF_TILE,), jnp.int32),
            pltpu.VMEM((EMBED_DIM,), jnp.float32),
            pltpu.VMEM((B_PER_W,), jnp.float32),
        ] + [pltpu.VMEM((IDX_PER_CHUNK,), jnp.float32)] * NBUF
          + [pltpu.SemaphoreType.DMA] * (NBUF + 1),
    )
    return run(x_flat, off_tile, t, b_vec)


# speedup vs baseline: 1.0026x; 1.0026x over previous
import jax
import jax.numpy as jnp
from jax import lax
from jax.experimental import pallas as pl
from jax.experimental.pallas import tpu as pltpu
from jax.experimental.pallas import tpu_sc as plsc

BATCH = 16384
N_FIELDS = 26
EMBED_DIM = 16
TABLE_ROWS = 1000000
T_PAD = 1000448

NC = 2
NS = 16
NW = NC * NS
B_PER_W = BATCH // NW
ITEMS_PER_CHUNK = 4
IDX_PER_CHUNK = ITEMS_PER_CHUNK * N_FIELDS
NCHUNKS = B_PER_W // ITEMS_PER_CHUNK
NBUF = 8
FLAT_PER_W = B_PER_W * N_FIELDS
OFF_TILE = 208
NVEC_OFF = OFF_TILE // EMBED_DIM
T_SLICE = T_PAD // NS

TC_BLK = 131072


def _tc_dot_kernel(tt_ref, w_ref, t_ref):
    t_ref[...] = jnp.sum(tt_ref[...] * (w_ref[...] * (1.0 / N_FIELDS)), axis=0)


def _sc_kernel(x_hbm, off_hbm, t_hbm, bias_hbm, out_hbm,
               t_sh, idx_v, off_v, b_v, acc_v,
               buf0, buf1, buf2, buf3, buf4, buf5, buf6, buf7,
               sem0, sem1, sem2, sem3, sem4, sem5, sem6, sem7, sem_stage):
    bufs = [buf0, buf1, buf2, buf3, buf4, buf5, buf6, buf7]
    sems = [sem0, sem1, sem2, sem3, sem4, sem5, sem6, sem7]

    sid = lax.axis_index("s")
    wid = sid * NC + lax.axis_index("c")
    base_flat = wid * FLAT_PER_W
    base_out = wid * B_PER_W

    sl = pl.ds(sid * T_SLICE, T_SLICE)
    stage_cp = pltpu.async_copy(t_hbm.at[sl], t_sh.at[sl], sem_stage)

    pltpu.sync_copy(x_hbm.at[pl.ds(base_flat, FLAT_PER_W)], idx_v)
    pltpu.sync_copy(off_hbm, off_v)
    pltpu.sync_copy(bias_hbm, b_v)

    off_regs = [off_v[pl.ds(k * EMBED_DIM, EMBED_DIM)] for k in range(NVEC_OFF)]

    def off_body(j, carry):
        p = j * OFF_TILE
        for k in range(NVEC_OFF):
            sl2 = pl.ds(p + k * EMBED_DIM, EMBED_DIM)
            idx_v[sl2] = idx_v[sl2] + off_regs[k]
        return carry

    lax.fori_loop(0, FLAT_PER_W // OFF_TILE, off_body, 0)

    stage_cp.wait()
    plsc.subcore_barrier()

    def gather_start(c, slot):
        idx_sl = idx_v.at[pl.ds(c * IDX_PER_CHUNK, IDX_PER_CHUNK)]
        pltpu.async_copy(t_sh.at[idx_sl], bufs[slot], sems[slot])

    def gather_wait(slot):
        idx_sl = idx_v.at[pl.ds(0, IDX_PER_CHUNK)]
        pltpu.make_async_copy(t_sh.at[idx_sl], bufs[slot], sems[slot]).wait()

    for b in range(NBUF):
        gather_start(b, b)

    lane_iota = lax.iota(jnp.int32, EMBED_DIM)
    WINDOWS = ((0, 0, 16, 10), (24, 2, 40, 12), (48, 4, 64, 14), (72, 6, 88, 16))

    ITEMS_PER_OUTER = NBUF * ITEMS_PER_CHUNK
    N_ACC = ITEMS_PER_OUTER // EMBED_DIM

    def outer(c0, carry):
        accs = [jnp.zeros((EMBED_DIM,), jnp.float32) for _ in range(N_ACC)]
        for b in range(NBUF):
            c = c0 * NBUF + b
            gather_wait(b)
            for item in range(ITEMS_PER_CHUNK):
                lo_off, lo_from, hi_off, hi_below = WINDOWS[item]
                v_lo = bufs[b][pl.ds(lo_off, EMBED_DIM)]
                v_hi = bufs[b][pl.ds(hi_off, EMBED_DIM)]
                if lo_from:
                    v_lo = jnp.where(lane_iota >= lo_from, v_lo, 0.0)
                if hi_below < EMBED_DIM:
                    v_hi = jnp.where(lane_iota < hi_below, v_hi, 0.0)
                v = v_lo + v_hi
                g = b * ITEMS_PER_CHUNK + item
                acc_i, lane = g // EMBED_DIM, g % EMBED_DIM
                accs[acc_i] = jnp.where(lane_iota == lane, jnp.sum(v), accs[acc_i])

            @pl.when(c + NBUF < NCHUNKS)
            def _():
                gather_start(c + NBUF, b)
        for i in range(N_ACC):
            acc_v[pl.ds(c0 * ITEMS_PER_OUTER + i * EMBED_DIM, EMBED_DIM)] = accs[i]
        return carry

    lax.fori_loop(0, NCHUNKS // NBUF, outer, 0)

    bv = b_v[...]

    def sig_body(v, carry):
        sl2 = pl.ds(v * EMBED_DIM, EMBED_DIM)
        z = acc_v[sl2] + bv
        acc_v[sl2] = 1.0 / (1.0 + jnp.exp(-z))
        return carry

    lax.fori_loop(0, B_PER_W // EMBED_DIM, sig_body, 0)

    pltpu.sync_copy(acc_v, out_hbm.at[pl.ds(base_out, B_PER_W)])


@jax.jit
def kernel(x, offsets, emb_table, W, b):
    x_flat = x.astype(jnp.int32).reshape(-1)
    off_tile = jnp.tile(offsets.astype(jnp.int32), OFF_TILE // N_FIELDS)
    b_vec = jnp.broadcast_to(b.astype(jnp.float32), (EMBED_DIM,))

    table_t = emb_table.T
    n_blk = (T_PAD + TC_BLK - 1) // TC_BLK
    t = pl.pallas_call(
        _tc_dot_kernel,
        grid=(n_blk,),
        in_specs=[
            pl.BlockSpec((EMBED_DIM, TC_BLK), lambda i: (0, i)),
            pl.BlockSpec((EMBED_DIM, 1), lambda i: (0, 0)),
        ],
        out_specs=pl.BlockSpec((TC_BLK,), lambda i: (i,)),
        out_shape=jax.ShapeDtypeStruct((T_PAD,), jnp.float32),
    )(table_t, W.astype(jnp.float32))

    mesh = plsc.VectorSubcoreMesh(core_axis_name="c", subcore_axis_name="s")
    run = pl.kernel(
        _sc_kernel,
        mesh=mesh,
        out_type=jax.ShapeDtypeStruct((BATCH,), jnp.float32),
        compiler_params=pltpu.CompilerParams(
            needs_layout_passes=False,
            use_tc_tiling_on_sc=False,
            skip_device_barrier=True,
        ),
        scratch_types=[
            pltpu.VMEM_SHARED((T_PAD,), jnp.float32),
            pltpu.VMEM((FLAT_PER_W,), jnp.int32),
            pltpu.VMEM((OFF_TILE,), jnp.int32),
            pltpu.VMEM((EMBED_DIM,), jnp.float32),
            pltpu.VMEM((B_PER_W,), jnp.float32),
        ] + [pltpu.VMEM((IDX_PER_CHUNK,), jnp.float32)] * NBUF
          + [pltpu.SemaphoreType.DMA] * (NBUF + 1),
    )
    return run(x_flat, off_tile, t, b_vec)
